# baseline (speedup 1.0000x reference)
import functools

import jax
import jax.numpy as jnp
from jax.experimental import pallas as pl
from jax.experimental.pallas import tpu as pltpu

N_ITEMS = 12464
INIT_DIM = 3072
HIDDEN = 1536
EMBED = 256
HIST = 200
TOPK = 20

ROWS_PER_STEP = 8
GATHER_STEPS = HIST // ROWS_PER_STEP

NB = 512
NBLK = (N_ITEMS + NB - 1) // NB
SCR_ROWS = ((NBLK + 7) // 8) * 8

_NEG_INF = float("-inf")


def _bf(x):
    return x.astype(jnp.bfloat16)


def _user_kernel(ints_ref, *refs):
    row_refs = refs[:ROWS_PER_STEP]
    w1_ref, b1_ref, w2_ref, b2_ref, out_ref, acc_ref = refs[ROWS_PER_STEP:]
    i = pl.program_id(0)

    @pl.when(i == 0)
    def _init():
        acc_ref[...] = jnp.zeros_like(acc_ref)

    s = None
    for j, r in enumerate(row_refs):
        m = ints_ref[ROWS_PER_STEP * i + j] % 8
        row = r[0, pl.ds(m, 1), :]
        s = row if s is None else s + row
    acc_ref[...] += s

    @pl.when(i == GATHER_STEPS - 1)
    def _finish():
        u = acc_ref[...] / jnp.float32(HIST)
        h = jnp.dot(_bf(u), w1_ref[...],
                    preferred_element_type=jnp.float32) + b1_ref[...]
        h = jnp.where(h > 0, h, jnp.float32(0.01) * h)
        e = jnp.dot(_bf(h), w2_ref[...],
                    preferred_element_type=jnp.float32) + b2_ref[...]
        n = jnp.sqrt(jnp.sum(e * e, axis=-1, keepdims=True))
        e = e / jnp.maximum(n, jnp.float32(1e-12))
        out_ref[...] = _bf(e)


def _items_kernel(x_ref, w1_ref, b1_ref, w2_ref, b2_ref, ub_ref,
                  out_ref, scr_ref):
    i = pl.program_id(0)
    x = _bf(x_ref[...])
    h = jnp.dot(x, w1_ref[...],
                preferred_element_type=jnp.float32) + b1_ref[...]
    h = jnp.where(h > 0, h, jnp.float32(0.01) * h)
    e = jnp.dot(_bf(h), w2_ref[...],
                preferred_element_type=jnp.float32) + b2_ref[...]
    n = jnp.sqrt(jnp.sum(e * e, axis=-1, keepdims=True))
    en = _bf(e / jnp.maximum(n, jnp.float32(1e-12)))
    s = jax.lax.dot_general(ub_ref[...], en, (((1,), (1,)), ((), ())),
                            preferred_element_type=jnp.float32)
    scr_ref[pl.ds(i, 1), :] = s

    @pl.when(i == NBLK - 1)
    def _topk():
        r_iota = jax.lax.broadcasted_iota(jnp.int32, (SCR_ROWS, NB), 0)
        c_iota = jax.lax.broadcasted_iota(jnp.int32, (SCR_ROWS, NB), 1)
        gidx = r_iota * NB + c_iota
        valid = gidx < N_ITEMS
        scores = jnp.where(valid, scr_ref[...], _NEG_INF)
        lane = jax.lax.broadcasted_iota(jnp.int32, (1, TOPK), 1)
        res0 = jnp.zeros((1, TOPK), jnp.int32)

        def body(k, carry):
            sc, res = carry
            m = jnp.max(sc)
            cand = jnp.where(sc == m, gidx, jnp.int32(2**30))
            g = jnp.min(cand)
            res = jnp.where(lane == k, g, res)
            sc = jnp.where(gidx == g, _NEG_INF, sc)
            return sc, res

        _, res = jax.lax.fori_loop(0, TOPK, body, (scores, res0))
        out_ref[...] = res


def kernel(interactions, item_cf_embeds, W1, b1, W2, b2):
    w1b = W1.astype(jnp.bfloat16)
    w2b = W2.astype(jnp.bfloat16)
    b1r = b1.reshape(1, HIDDEN)
    b2r = b2.reshape(1, EMBED)
    ints = interactions.astype(jnp.int32)

    xg = item_cf_embeds.reshape(N_ITEMS // 8, 8, INIT_DIM)
    row_spec = [
        pl.BlockSpec((1, 8, INIT_DIM),
                     functools.partial(
                         lambda j, i, ints_ref: (ints_ref[ROWS_PER_STEP * i + j] // 8, 0, 0), j))
        for j in range(ROWS_PER_STEP)
    ]
    const2 = lambda i, ints_ref: (0, 0)
    ub = pl.pallas_call(
        _user_kernel,
        grid_spec=pltpu.PrefetchScalarGridSpec(
            num_scalar_prefetch=1,
            grid=(GATHER_STEPS,),
            in_specs=row_spec + [
                pl.BlockSpec((INIT_DIM, HIDDEN), const2),
                pl.BlockSpec((1, HIDDEN), const2),
                pl.BlockSpec((HIDDEN, EMBED), const2),
                pl.BlockSpec((1, EMBED), const2),
            ],
            out_specs=pl.BlockSpec((1, EMBED), const2),
            scratch_shapes=[pltpu.VMEM((1, INIT_DIM), jnp.float32)],
        ),
        out_shape=jax.ShapeDtypeStruct((1, EMBED), jnp.bfloat16),
    )(ints, *([xg] * ROWS_PER_STEP), w1b, b1r, w2b, b2r)

    cst = lambda i: (0, 0)
    idx = pl.pallas_call(
        _items_kernel,
        grid=(NBLK,),
        in_specs=[
            pl.BlockSpec((NB, INIT_DIM), lambda i: (i, 0)),
            pl.BlockSpec((INIT_DIM, HIDDEN), cst),
            pl.BlockSpec((1, HIDDEN), cst),
            pl.BlockSpec((HIDDEN, EMBED), cst),
            pl.BlockSpec((1, EMBED), cst),
            pl.BlockSpec((1, EMBED), cst),
        ],
        out_specs=pl.BlockSpec((1, TOPK), cst),
        out_shape=jax.ShapeDtypeStruct((1, TOPK), jnp.int32),
        scratch_shapes=[pltpu.VMEM((SCR_ROWS, NB), jnp.float32)],
    )(item_cf_embeds, w1b, b1r, w2b, b2r, ub)
    return idx


# speedup vs baseline: 1.0983x; 1.0983x over previous
import functools

import jax
import jax.numpy as jnp
from jax.experimental import pallas as pl
from jax.experimental.pallas import tpu as pltpu

N_ITEMS = 12464
INIT_DIM = 3072
HIDDEN = 1536
EMBED = 256
HIST = 200
TOPK = 20

NB = 512
NBLK = (N_ITEMS + NB - 1) // NB
NPAD = NBLK * NB
ROWS_PER_STEP = HIST // NBLK

_NEG_INF = float("-inf")


def _bf(x):
    return x.astype(jnp.bfloat16)


def _leaky(h):
    return jnp.where(h > 0, h, jnp.float32(0.01) * h)


def _fused_kernel(ints_ref, *refs):
    grefs = refs[:ROWS_PER_STEP]
    (x_ref, w1_ref, b1_ref, w2_ref, b2_ref,
     out_ref, acc_ref, en_ref) = refs[ROWS_PER_STEP:]
    i = pl.program_id(0)

    @pl.when(i == 0)
    def _init():
        acc_ref[...] = jnp.zeros_like(acc_ref)

    s = None
    for j, r in enumerate(grefs):
        m = ints_ref[ROWS_PER_STEP * i + j] % 8
        row = r[0, pl.ds(m, 1), :]
        s = row if s is None else s + row
    acc_ref[...] += s

    x = _bf(x_ref[...])
    h = jnp.dot(x, w1_ref[...], preferred_element_type=jnp.float32) + b1_ref[...]
    h = _leaky(h)
    e = jnp.dot(_bf(h), w2_ref[...], preferred_element_type=jnp.float32) + b2_ref[...]
    n = jnp.sqrt(jnp.sum(e * e, axis=-1, keepdims=True))
    en_ref[pl.ds(i * NB, NB), :] = _bf(e / jnp.maximum(n, jnp.float32(1e-12)))

    @pl.when(i == NBLK - 1)
    def _finish():
        u = acc_ref[...] / jnp.float32(HIST)
        hu = _leaky(jnp.dot(_bf(u), w1_ref[...],
                            preferred_element_type=jnp.float32) + b1_ref[...])
        eu = jnp.dot(_bf(hu), w2_ref[...],
                     preferred_element_type=jnp.float32) + b2_ref[...]
        nu = jnp.sqrt(jnp.sum(eu * eu, axis=-1, keepdims=True))
        ub = _bf(eu / jnp.maximum(nu, jnp.float32(1e-12)))

        sc = jax.lax.dot_general(ub, en_ref[...], (((1,), (1,)), ((), ())),
                                 preferred_element_type=jnp.float32)
        sc = sc.reshape(NPAD // 128, 128)
        r_iota = jax.lax.broadcasted_iota(jnp.int32, (NPAD // 128, 128), 0)
        c_iota = jax.lax.broadcasted_iota(jnp.int32, (NPAD // 128, 128), 1)
        gidx = r_iota * 128 + c_iota
        sc = jnp.where(gidx < N_ITEMS, sc, _NEG_INF)
        lane = jax.lax.broadcasted_iota(jnp.int32, (1, TOPK), 1)

        def body(k, carry):
            scv, res = carry
            mv = jnp.max(scv)
            g = jnp.min(jnp.where(scv == mv, gidx, jnp.int32(2**30)))
            res = jnp.where(lane == k, g, res)
            scv = jnp.where(gidx == g, _NEG_INF, scv)
            return scv, res

        _, res = jax.lax.fori_loop(0, TOPK, body,
                                   (sc, jnp.zeros((1, TOPK), jnp.int32)))
        out_ref[...] = res


def kernel(interactions, item_cf_embeds, W1, b1, W2, b2):
    w1b = W1.astype(jnp.bfloat16)
    w2b = W2.astype(jnp.bfloat16)
    b1r = b1.reshape(1, HIDDEN)
    b2r = b2.reshape(1, EMBED)
    ints = interactions.astype(jnp.int32)
    xg = item_cf_embeds.reshape(N_ITEMS // 8, 8, INIT_DIM)

    gather_spec = [
        pl.BlockSpec((1, 8, INIT_DIM),
                     functools.partial(
                         lambda j, i, ir: (ir[ROWS_PER_STEP * i + j] // 8, 0, 0), j))
        for j in range(ROWS_PER_STEP)
    ]
    const2 = lambda i, ir: (0, 0)
    idx = pl.pallas_call(
        _fused_kernel,
        grid_spec=pltpu.PrefetchScalarGridSpec(
            num_scalar_prefetch=1,
            grid=(NBLK,),
            in_specs=gather_spec + [
                pl.BlockSpec((NB, INIT_DIM), lambda i, ir: (i, 0)),
                pl.BlockSpec((INIT_DIM, HIDDEN), const2),
                pl.BlockSpec((1, HIDDEN), const2),
                pl.BlockSpec((HIDDEN, EMBED), const2),
                pl.BlockSpec((1, EMBED), const2),
            ],
            out_specs=pl.BlockSpec((1, TOPK), const2),
            scratch_shapes=[pltpu.VMEM((1, INIT_DIM), jnp.float32),
                            pltpu.VMEM((NPAD, EMBED), jnp.bfloat16)],
        ),
        out_shape=jax.ShapeDtypeStruct((1, TOPK), jnp.int32),
    )(ints, *([xg] * ROWS_PER_STEP), item_cf_embeds, w1b, b1r, w2b, b2r)
    return idx


# speedup vs baseline: 1.1561x; 1.0527x over previous
import functools

import jax
import jax.numpy as jnp
from jax.experimental import pallas as pl
from jax.experimental.pallas import tpu as pltpu

N_ITEMS = 12464
INIT_DIM = 3072
HIDDEN = 1536
EMBED = 256
HIST = 200
TOPK = 20

NB = 512
NBLK = (N_ITEMS + NB - 1) // NB
NPAD = NBLK * NB
ROWS_PER_STEP = HIST // NBLK

_NEG_INF = float("-inf")


def _leaky(h):
    return jnp.where(h > 0, h, jnp.float32(0.01) * h)


def _fused_kernel(ints_ref, *refs):
    grefs = refs[:ROWS_PER_STEP]
    (x_ref, w1_ref, b1_ref, w2_ref, b2_ref,
     out_ref, acc_ref, en_ref) = refs[ROWS_PER_STEP:]
    i = pl.program_id(0)

    @pl.when(i == 0)
    def _init():
        acc_ref[...] = jnp.zeros_like(acc_ref)

    s = None
    for j, r in enumerate(grefs):
        m = ints_ref[ROWS_PER_STEP * i + j] % 8
        row = r[0, pl.ds(m, 1), :]
        s = row if s is None else s + row
    acc_ref[...] += s

    x = x_ref[...]
    h = jnp.dot(x, w1_ref[...], preferred_element_type=jnp.float32) + b1_ref[...]
    h = _leaky(h)
    e = jnp.dot(h, w2_ref[...], preferred_element_type=jnp.float32) + b2_ref[...]
    n = jnp.sqrt(jnp.sum(e * e, axis=-1, keepdims=True))
    en_ref[pl.ds(i * NB, NB), :] = e / jnp.maximum(n, jnp.float32(1e-12))

    @pl.when(i == NBLK - 1)
    def _finish():
        u = acc_ref[...] / jnp.float32(HIST)
        hu = _leaky(jnp.dot(u, w1_ref[...],
                            preferred_element_type=jnp.float32) + b1_ref[...])
        eu = jnp.dot(hu, w2_ref[...],
                     preferred_element_type=jnp.float32) + b2_ref[...]
        nu = jnp.sqrt(jnp.sum(eu * eu, axis=-1, keepdims=True))
        ub = eu / jnp.maximum(nu, jnp.float32(1e-12))

        sc = jax.lax.dot_general(ub, en_ref[...], (((1,), (1,)), ((), ())),
                                 preferred_element_type=jnp.float32)
        sc = sc.reshape(NPAD // 128, 128)
        r_iota = jax.lax.broadcasted_iota(jnp.int32, (NPAD // 128, 128), 0)
        c_iota = jax.lax.broadcasted_iota(jnp.int32, (NPAD // 128, 128), 1)
        gidx = r_iota * 128 + c_iota
        sc = jnp.where(gidx < N_ITEMS, sc, _NEG_INF)
        lane = jax.lax.broadcasted_iota(jnp.int32, (1, TOPK), 1)

        def body(k, carry):
            scv, res = carry
            mv = jnp.max(scv)
            g = jnp.min(jnp.where(scv == mv, gidx, jnp.int32(2**30)))
            res = jnp.where(lane == k, g, res)
            scv = jnp.where(gidx == g, _NEG_INF, scv)
            return scv, res

        _, res = jax.lax.fori_loop(0, TOPK, body,
                                   (sc, jnp.zeros((1, TOPK), jnp.int32)))
        out_ref[...] = res


def kernel(interactions, item_cf_embeds, W1, b1, W2, b2):
    b1r = b1.reshape(1, HIDDEN)
    b2r = b2.reshape(1, EMBED)
    ints = interactions.astype(jnp.int32)
    xg = item_cf_embeds.reshape(N_ITEMS // 8, 8, INIT_DIM)

    gather_spec = [
        pl.BlockSpec((1, 8, INIT_DIM),
                     functools.partial(
                         lambda j, i, ir: (ir[ROWS_PER_STEP * i + j] // 8, 0, 0), j))
        for j in range(ROWS_PER_STEP)
    ]
    const2 = lambda i, ir: (0, 0)
    idx = pl.pallas_call(
        _fused_kernel,
        grid_spec=pltpu.PrefetchScalarGridSpec(
            num_scalar_prefetch=1,
            grid=(NBLK,),
            in_specs=gather_spec + [
                pl.BlockSpec((NB, INIT_DIM), lambda i, ir: (i, 0)),
                pl.BlockSpec((INIT_DIM, HIDDEN), const2),
                pl.BlockSpec((1, HIDDEN), const2),
                pl.BlockSpec((HIDDEN, EMBED), const2),
                pl.BlockSpec((1, EMBED), const2),
            ],
            out_specs=pl.BlockSpec((1, TOPK), const2),
            scratch_shapes=[pltpu.VMEM((1, INIT_DIM), jnp.float32),
                            pltpu.VMEM((NPAD, EMBED), jnp.float32)],
        ),
        out_shape=jax.ShapeDtypeStruct((1, TOPK), jnp.int32),
    )(ints, *([xg] * ROWS_PER_STEP), item_cf_embeds, W1, b1r, W2, b2r)
    return idx
